# Initial kernel scaffold; baseline (speedup 1.0000x reference)
import jax
import jax.numpy as jnp
from jax.experimental import pallas as pl


def kernel(x, edge_index, edge_weight, W1, b1, W2, b2):
    raise NotImplementedError("write your pallas kernel here")


# speedup vs baseline: 1.0816x; 1.0816x over previous
import functools

import jax
import jax.numpy as jnp
from jax.experimental import pallas as pl
from jax.experimental.pallas import tpu as pltpu

_N = 10000
_F = 128
_H = 128
_D = 64
_BR = 400


def _mm_bias_kernel(x_ref, w_ref, b_ref, o_ref):
    o_ref[...] = (
        jnp.dot(x_ref[...], w_ref[...], preferred_element_type=jnp.float32)
        + b_ref[...][None, :]
    )


def _dense(x, W, b):
    return pl.pallas_call(
        _mm_bias_kernel,
        out_shape=jax.ShapeDtypeStruct((x.shape[0], W.shape[1]), jnp.float32),
    )(x, W, b)


def _decode_kernel(zr_ref, zf_ref, adj_ref):
    acc = jax.lax.dot_general(
        zr_ref[...], zf_ref[...],
        (((1,), (1,)), ((), ())),
        preferred_element_type=jnp.float32,
    )
    adj_ref[...] = jax.nn.sigmoid(acc)


def _decode(z):
    return pl.pallas_call(
        _decode_kernel,
        grid=(_N // _BR,),
        in_specs=[
            pl.BlockSpec((_BR, _D), lambda i: (i, 0)),
            pl.BlockSpec((_N, _D), lambda i: (0, 0)),
        ],
        out_specs=pl.BlockSpec((_BR, _N), lambda i: (i, 0)),
        out_shape=jax.ShapeDtypeStruct((_N, _N), jnp.float32),
    )(z, z)


def _spmm(edge_index, edge_weight, support):
    dst = edge_index[0]
    src = edge_index[1]
    msgs = jnp.take(support, src, axis=0) * edge_weight[:, None]
    return jax.ops.segment_sum(msgs, dst, num_segments=_N)


def kernel(x, edge_index, edge_weight, W1, b1, W2, b2):
    support1 = _dense(x, W1, b1)
    h = jax.nn.relu(_spmm(edge_index, edge_weight, support1))
    support2 = _dense(h, W2, b2)
    z = _spmm(edge_index, edge_weight, support2)
    adj_rec = _decode(z)
    return (z, adj_rec)


# speedup vs baseline: 2.8139x; 2.6017x over previous
import functools

import jax
import jax.numpy as jnp
from jax import lax
from jax.experimental import pallas as pl
from jax.experimental.pallas import tpu as pltpu
from jax.experimental.pallas import tpu_sc as plsc

_N = 10000
_E = 160000
_F = 128
_H = 128
_D = 64
_BR = 400
_CH = 128
_NCH = 40
_NSC = 2
_NSUB = 16
_EPAD = _NSC * _NSUB * _NCH * _CH
_NACC = 10240
_NZ = _NACC // _NSUB


def _mm_bias_kernel(x_ref, w_ref, b_ref, o_ref):
    o_ref[...] = (
        jnp.dot(x_ref[...], w_ref[...], preferred_element_type=jnp.float32)
        + b_ref[...][None, :]
    )


def _dense(x, W, b):
    return pl.pallas_call(
        _mm_bias_kernel,
        out_shape=jax.ShapeDtypeStruct((x.shape[0], W.shape[1]), jnp.float32),
    )(x, W, b)


def _fuse2_kernel(p_ref, w_ref, b_ref, o_ref):
    h = jax.nn.relu(p_ref[0, :_N] + p_ref[1, :_N])
    o_ref[...] = (
        jnp.dot(h, w_ref[...], preferred_element_type=jnp.float32)
        + b_ref[...][None, :]
    )


def _fuse2(p, W, b):
    return pl.pallas_call(
        _fuse2_kernel,
        out_shape=jax.ShapeDtypeStruct((_N, W.shape[1]), jnp.float32),
    )(p, W, b)


def _zadd_kernel(q_ref, z_ref):
    z_ref[...] = q_ref[0, :_N, :_D] + q_ref[1, :_N, :_D]


def _zadd(q):
    return pl.pallas_call(
        _zadd_kernel,
        out_shape=jax.ShapeDtypeStruct((_N, _D), jnp.float32),
    )(q)


def _decode_kernel(zr_ref, zf_ref, adj_ref):
    acc = jax.lax.dot_general(
        zr_ref[...], zf_ref[...],
        (((1,), (1,)), ((), ())),
        preferred_element_type=jnp.float32,
    )
    adj_ref[...] = jax.nn.sigmoid(acc)


def _decode(z):
    return pl.pallas_call(
        _decode_kernel,
        grid=(_N // _BR,),
        in_specs=[
            pl.BlockSpec((_BR, _D), lambda i: (i, 0)),
            pl.BlockSpec((_N, _D), lambda i: (0, 0)),
        ],
        out_specs=pl.BlockSpec((_BR, _N), lambda i: (i, 0)),
        out_shape=jax.ShapeDtypeStruct((_N, _N), jnp.float32),
    )(z, z)


def _make_spmm_sc(F):
    mesh = plsc.VectorSubcoreMesh(core_axis_name="c", subcore_axis_name="s")

    @functools.partial(
        pl.kernel,
        out_type=jax.ShapeDtypeStruct((_NSC, _NACC, F), jnp.float32),
        mesh=mesh,
        scratch_types=[
            pltpu.VMEM((_NCH, _CH), jnp.int32),
            pltpu.VMEM((_NCH, _CH), jnp.int32),
            pltpu.VMEM((_NCH, _CH), jnp.float32),
            pltpu.VMEM((_CH, F), jnp.float32),
            pltpu.VMEM_SHARED((_NACC, F), jnp.float32),
            pltpu.SemaphoreType.DMA,
        ],
    )
    def spmm(src_hbm, dst_hbm, w_hbm, sup_hbm, zero_hbm, out_hbm,
             src_v, dst_v, w_v, buf, acc, sem):
        c = lax.axis_index("c")
        s = lax.axis_index("s")
        pltpu.sync_copy(src_hbm.at[c, s], src_v)
        pltpu.sync_copy(dst_hbm.at[c, s], dst_v)
        pltpu.sync_copy(w_hbm.at[c, s], w_v)
        pltpu.sync_copy(zero_hbm, acc.at[pl.ds(s * _NZ, _NZ)])
        plsc.subcore_barrier()

        @pl.loop(0, _NCH)
        def _chunks(k):
            pltpu.async_copy(sup_hbm.at[src_v.at[k]], buf, sem).wait()

            @pl.loop(0, _CH // 16)
            def _groups(gi):
                w16 = w_v[k, pl.ds(gi * 16, 16)]
                for j in range(16):
                    wj = w16[j]
                    e = gi * 16 + j
                    for g in range(F // 16):
                        buf[e, pl.ds(g * 16, 16)] = buf[e, pl.ds(g * 16, 16)] * wj

            pltpu.sync_copy(buf, acc.at[dst_v.at[k]], add=True)

        plsc.subcore_barrier()
        pltpu.sync_copy(acc.at[pl.ds(s * _NZ, _NZ)],
                        out_hbm.at[c, pl.ds(s * _NZ, _NZ)])

    return spmm


_spmm128 = _make_spmm_sc(_F)


def kernel(x, edge_index, edge_weight, W1, b1, W2, b2):
    pad = _EPAD - _E
    srcp = jnp.pad(edge_index[1], (0, pad)).reshape(_NSC, _NSUB, _NCH, _CH)
    dstp = jnp.pad(edge_index[0], (0, pad)).reshape(_NSC, _NSUB, _NCH, _CH)
    wp = jnp.pad(edge_weight, (0, pad)).reshape(_NSC, _NSUB, _NCH, _CH)
    zero_h = jnp.zeros((_NZ, _H), jnp.float32)
    W2p = jnp.pad(W2, ((0, 0), (0, _F - _D)))
    b2p = jnp.pad(b2, (0, _F - _D))

    support1 = _dense(x, W1, b1)
    p = _spmm128(srcp, dstp, wp, support1, zero_h)
    support2 = _fuse2(p, W2p, b2p)
    q = _spmm128(srcp, dstp, wp, support2, zero_h)
    z = _zadd(q)
    adj_rec = _decode(z)
    return (z, adj_rec)
